# baseline (speedup 1.0000x reference)
import functools

import jax
import jax.numpy as jnp
from jax import lax
from jax.experimental import pallas as pl
from jax.experimental.pallas import tpu as pltpu
from jax.experimental.pallas import tpu_sc as plsc

E = 320000
D = 128
NC = 2
NS = 16
NW = NC * NS
ROWS_PER_W = E // NW
CB = 400
NCHUNK = ROWS_PER_W // CB

_mesh = plsc.VectorSubcoreMesh(core_axis_name="c", subcore_axis_name="s")


@functools.partial(
    pl.kernel,
    out_type=jax.ShapeDtypeStruct((E, D), jnp.float32),
    mesh=_mesh,
    scratch_types=[
        pltpu.VMEM((5, D), jnp.float32),
        pltpu.VMEM((6, D), jnp.float32),
        pltpu.VMEM((2, D), jnp.float32),
        pltpu.VMEM((8, D), jnp.float32),
        pltpu.VMEM_SHARED((64, D), jnp.float32),
        pltpu.VMEM((CB,), jnp.int32),
        pltpu.VMEM((CB,), jnp.int32),
        pltpu.VMEM((CB,), jnp.int32),
        pltpu.VMEM((CB,), jnp.int32),
        pltpu.VMEM((CB,), jnp.int32),
        pltpu.VMEM((CB,), jnp.int32),
        pltpu.VMEM((CB,), jnp.int32),
        pltpu.VMEM((CB,), jnp.int32),
        pltpu.VMEM((2, CB, D), jnp.float32),
        pltpu.SemaphoreType.DMA,
        pltpu.SemaphoreType.DMA,
        pltpu.SemaphoreType.DMA,
        pltpu.SemaphoreType.DMA,
        pltpu.SemaphoreType.DMA,
        pltpu.SemaphoreType.DMA,
    ],
)
def _bond_encoder_sc(a0_hbm, a1_hbm, a2_hbm, w0_hbm, w1_hbm, w2_hbm, out_hbm,
                     w0_v, w1_v, w2_v, combo_v, combo_sh,
                     a00, a01, a10, a11, a20, a21, idx_a, idx_b, out_v,
                     si0, si1, so0, so1, sg0, sg1):
    sid = lax.axis_index("s")
    wid = sid * NC + lax.axis_index("c")
    base = wid * ROWS_PER_W
    cols = ((a0_hbm, (a00, a01)), (a1_hbm, (a10, a11)), (a2_hbm, (a20, a21)))
    idxs = (idx_a, idx_b)
    sin = (si0, si1)
    sout = (so0, so1)
    sg = (sg0, sg1)


    def start_in(g, b):
        for hbm, v in cols:
            pltpu.async_copy(hbm.at[pl.ds(base + g * CB, CB)],
                             v[b], sin[b])

    def wait_in(b):
        for hbm, v in cols:
            pltpu.make_async_copy(hbm.at[pl.ds(0, CB)], v[b],
                                  sin[b]).wait()

    H = CB // 2

    def start_out(g, b):
        off = base + g * CB
        pltpu.async_copy(out_v.at[b, pl.ds(0, H)],
                         out_hbm.at[pl.ds(off, H)], sout[b])
        pltpu.async_copy(out_v.at[b, pl.ds(H, H)],
                         out_hbm.at[pl.ds(off + H, H)], sout[b])

    def wait_out(b):
        for _i in range(2):
            pltpu.make_async_copy(out_v.at[b, pl.ds(0, H)],
                                  out_hbm.at[pl.ds(0, H)], sout[b]).wait()

    def comp_idx(b):
        @plsc.parallel_loop(0, CB // 16)
        def _t(t):
            sl = pl.ds(16 * t, 16)
            idxs[b][sl] = (cols[0][1][b][sl] * 12 + cols[1][1][b][sl] * 2
                           + cols[2][1][b][sl])

    def start_gather(b):
        pltpu.async_copy(combo_sh.at[idxs[b]], out_v.at[b], sg[b])

    def wait_gather(b):
        pltpu.make_async_copy(combo_sh.at[idxs[b]], out_v.at[b],
                              sg[b]).wait()

    start_in(0, 0)
    start_in(1, 1)

    @pl.when(sid < 8)
    def _build():
        pltpu.sync_copy(w0_hbm, w0_v)
        pltpu.sync_copy(w1_hbm, w1_v)
        pltpu.sync_copy(w2_hbm, w2_v)

        def build_combo(r, _):
            c = jnp.minimum(8 * sid + r, 59)
            i = c // 12
            rr = c - i * 12
            j = rr // 2
            k = rr - j * 2
            for v in range(8):
                sl = pl.ds(16 * v, 16)
                combo_v[r, sl] = w0_v[i, sl] + w1_v[j, sl] + w2_v[k, sl]
            return _

        lax.fori_loop(0, 8, build_combo, None)
        pltpu.sync_copy(combo_v, combo_sh.at[pl.ds(8 * sid, 8)])

    wait_in(0)
    comp_idx(0)
    plsc.subcore_barrier()
    start_gather(0)

    def pair_body(p, _):
        wait_in(1)
        comp_idx(1)

        @pl.when(p >= 1)
        def _():
            wait_out(1)

        start_gather(1)
        wait_gather(0)
        start_out(2 * p, 0)
        start_in(2 * p + 2, 0)

        wait_in(0)
        comp_idx(0)
        wait_out(0)
        start_gather(0)
        wait_gather(1)
        start_out(2 * p + 1, 1)

        @pl.when(p < 11)
        def _():
            start_in(2 * p + 3, 1)
        return _

    lax.fori_loop(0, (NCHUNK - 1) // 2, pair_body, None)

    wait_gather(0)
    start_out(NCHUNK - 1, 0)
    wait_out(1)
    wait_out(0)


def kernel(edge_attr, W0, W1, W2):
    ea = edge_attr.astype(jnp.int32)
    return _bond_encoder_sc(ea[:, 0], ea[:, 1], ea[:, 2], W0, W1, W2)


# speedup vs baseline: 1.0021x; 1.0021x over previous
import functools

import jax
import jax.numpy as jnp
from jax import lax
from jax.experimental import pallas as pl
from jax.experimental.pallas import tpu as pltpu
from jax.experimental.pallas import tpu_sc as plsc

E = 320000
D = 128
NC = 2
NS = 16
NW = NC * NS
ROWS_PER_W = E // NW
CB = 400
NCHUNK = ROWS_PER_W // CB

_mesh = plsc.VectorSubcoreMesh(core_axis_name="c", subcore_axis_name="s")


@functools.partial(
    pl.kernel,
    out_type=jax.ShapeDtypeStruct((E, D), jnp.float32),
    mesh=_mesh,
    scratch_types=[
        pltpu.VMEM((5, D), jnp.float32),
        pltpu.VMEM((6, D), jnp.float32),
        pltpu.VMEM((2, D), jnp.float32),
        pltpu.VMEM((8, D), jnp.float32),
        pltpu.VMEM_SHARED((64, D), jnp.float32),
        pltpu.VMEM((CB,), jnp.int32),
        pltpu.VMEM((CB,), jnp.int32),
        pltpu.VMEM((CB,), jnp.int32),
        pltpu.VMEM((CB,), jnp.int32),
        pltpu.VMEM((CB,), jnp.int32),
        pltpu.VMEM((CB,), jnp.int32),
        pltpu.VMEM((CB,), jnp.int32),
        pltpu.VMEM((CB,), jnp.int32),
        pltpu.VMEM((2, CB, D), jnp.float32),
        pltpu.SemaphoreType.DMA,
        pltpu.SemaphoreType.DMA,
        pltpu.SemaphoreType.DMA,
        pltpu.SemaphoreType.DMA,
        pltpu.SemaphoreType.DMA,
        pltpu.SemaphoreType.DMA,
    ],
)
def _bond_encoder_sc(a0_hbm, a1_hbm, a2_hbm, w0_hbm, w1_hbm, w2_hbm, out_hbm,
                     w0_v, w1_v, w2_v, combo_v, combo_sh,
                     a00, a01, a10, a11, a20, a21, idx_a, idx_b, out_v,
                     si0, si1, so0, so1, sg0, sg1):
    sid = lax.axis_index("s")
    wid = sid * NC + lax.axis_index("c")
    base = wid * ROWS_PER_W
    cols = ((a0_hbm, (a00, a01)), (a1_hbm, (a10, a11)), (a2_hbm, (a20, a21)))
    idxs = (idx_a, idx_b)
    sin = (si0, si1)
    sout = (so0, so1)
    sg = (sg0, sg1)


    def start_in(g, b):
        for hbm, v in cols:
            pltpu.async_copy(hbm.at[pl.ds(base + g * CB, CB)],
                             v[b], sin[b])

    def wait_in(b):
        for hbm, v in cols:
            pltpu.make_async_copy(hbm.at[pl.ds(0, CB)], v[b],
                                  sin[b]).wait()

    def start_out(g, b):
        pltpu.async_copy(out_v.at[b], out_hbm.at[pl.ds(base + g * CB, CB)],
                         sout[b])

    def wait_out(b):
        pltpu.make_async_copy(out_v.at[b], out_hbm.at[pl.ds(0, CB)],
                              sout[b]).wait()

    def comp_idx(b):
        @plsc.parallel_loop(0, CB // 16)
        def _t(t):
            sl = pl.ds(16 * t, 16)
            idxs[b][sl] = (cols[0][1][b][sl] * 12 + cols[1][1][b][sl] * 2
                           + cols[2][1][b][sl])

    GSUB = 80

    def start_gather(b):
        for k in range(CB // GSUB):
            pltpu.async_copy(combo_sh.at[idxs[b].at[pl.ds(k * GSUB, GSUB)]],
                             out_v.at[b, pl.ds(k * GSUB, GSUB)], sg[b])

    def wait_gather(b):
        for k in range(CB // GSUB):
            pltpu.make_async_copy(combo_sh.at[idxs[b].at[pl.ds(0, GSUB)]],
                                  out_v.at[b, pl.ds(0, GSUB)], sg[b]).wait()

    start_in(0, 0)
    start_in(1, 1)

    @pl.when(sid < 8)
    def _build():
        pltpu.sync_copy(w0_hbm, w0_v)
        pltpu.sync_copy(w1_hbm, w1_v)
        pltpu.sync_copy(w2_hbm, w2_v)

        def build_combo(r, _):
            c = jnp.minimum(8 * sid + r, 59)
            i = c // 12
            rr = c - i * 12
            j = rr // 2
            k = rr - j * 2
            for v in range(8):
                sl = pl.ds(16 * v, 16)
                combo_v[r, sl] = w0_v[i, sl] + w1_v[j, sl] + w2_v[k, sl]
            return _

        lax.fori_loop(0, 8, build_combo, None)
        pltpu.sync_copy(combo_v, combo_sh.at[pl.ds(8 * sid, 8)])

    wait_in(0)
    comp_idx(0)
    plsc.subcore_barrier()
    start_gather(0)

    def pair_body(p, _):
        wait_in(1)
        comp_idx(1)

        @pl.when(p >= 1)
        def _():
            wait_out(1)

        start_gather(1)
        wait_gather(0)
        start_out(2 * p, 0)
        start_in(2 * p + 2, 0)

        wait_in(0)
        comp_idx(0)
        wait_out(0)
        start_gather(0)
        wait_gather(1)
        start_out(2 * p + 1, 1)

        @pl.when(p < 11)
        def _():
            start_in(2 * p + 3, 1)
        return _

    lax.fori_loop(0, (NCHUNK - 1) // 2, pair_body, None)

    wait_gather(0)
    start_out(NCHUNK - 1, 0)
    wait_out(1)
    wait_out(0)


def kernel(edge_attr, W0, W1, W2):
    ea = edge_attr.astype(jnp.int32)
    return _bond_encoder_sc(ea[:, 0], ea[:, 1], ea[:, 2], W0, W1, W2)


# speedup vs baseline: 1.0114x; 1.0093x over previous
import functools

import jax
import jax.numpy as jnp
from jax import lax
from jax.experimental import pallas as pl
from jax.experimental.pallas import tpu as pltpu
from jax.experimental.pallas import tpu_sc as plsc

E = 320000
D = 128
NC = 2
NS = 16
NW = NC * NS
ROWS_PER_W = E // NW
CB = 400
NCHUNK = ROWS_PER_W // CB

_mesh = plsc.VectorSubcoreMesh(core_axis_name="c", subcore_axis_name="s")


@functools.partial(
    pl.kernel,
    out_type=jax.ShapeDtypeStruct((E, D), jnp.float32),
    mesh=_mesh,
    scratch_types=[
        pltpu.VMEM((5, D), jnp.float32),
        pltpu.VMEM((6, D), jnp.float32),
        pltpu.VMEM((2, D), jnp.float32),
        pltpu.VMEM((8, D), jnp.float32),
        pltpu.VMEM_SHARED((64, D), jnp.float32),
        pltpu.VMEM((CB,), jnp.int32),
        pltpu.VMEM((CB,), jnp.int32),
        pltpu.VMEM((CB,), jnp.int32),
        pltpu.VMEM((CB,), jnp.int32),
        pltpu.VMEM((CB,), jnp.int32),
        pltpu.VMEM((CB,), jnp.int32),
        pltpu.VMEM((CB,), jnp.int32),
        pltpu.VMEM((CB,), jnp.int32),
        pltpu.VMEM((2, CB, D), jnp.float32),
        pltpu.SemaphoreType.DMA,
        pltpu.SemaphoreType.DMA,
        pltpu.SemaphoreType.DMA,
        pltpu.SemaphoreType.DMA,
        pltpu.SemaphoreType.DMA,
        pltpu.SemaphoreType.DMA,
        pltpu.SemaphoreType.DMA,
    ],
)
def _bond_encoder_sc(a0_hbm, a1_hbm, a2_hbm, w0_hbm, w1_hbm, w2_hbm, out_hbm,
                     w0_v, w1_v, w2_v, combo_v, combo_sh,
                     a00, a01, a10, a11, a20, a21, idx_a, idx_b, out_v,
                     si0, si1, so0, so1, sg0, sg1, sw):
    sid = lax.axis_index("s")
    wid = sid * NC + lax.axis_index("c")
    base = wid * ROWS_PER_W
    cols = ((a0_hbm, (a00, a01)), (a1_hbm, (a10, a11)), (a2_hbm, (a20, a21)))
    idxs = (idx_a, idx_b)
    sin = (si0, si1)
    sout = (so0, so1)
    sg = (sg0, sg1)


    def start_in(g, b):
        for hbm, v in cols:
            pltpu.async_copy(hbm.at[pl.ds(base + g * CB, CB)],
                             v[b], sin[b])

    def wait_in(b):
        for hbm, v in cols:
            pltpu.make_async_copy(hbm.at[pl.ds(0, CB)], v[b],
                                  sin[b]).wait()

    def start_out(g, b):
        pltpu.async_copy(out_v.at[b], out_hbm.at[pl.ds(base + g * CB, CB)],
                         sout[b])

    def wait_out(b):
        pltpu.make_async_copy(out_v.at[b], out_hbm.at[pl.ds(0, CB)],
                              sout[b]).wait()

    def comp_idx(b):
        @plsc.parallel_loop(0, CB // 16)
        def _t(t):
            sl = pl.ds(16 * t, 16)
            idxs[b][sl] = (cols[0][1][b][sl] * 12 + cols[1][1][b][sl] * 2
                           + cols[2][1][b][sl])

    GSUB = 80

    def start_gather(b):
        for k in range(CB // GSUB):
            pltpu.async_copy(combo_sh.at[idxs[b].at[pl.ds(k * GSUB, GSUB)]],
                             out_v.at[b, pl.ds(k * GSUB, GSUB)], sg[b])

    def wait_gather(b):
        for k in range(CB // GSUB):
            pltpu.make_async_copy(combo_sh.at[idxs[b].at[pl.ds(0, GSUB)]],
                                  out_v.at[b, pl.ds(0, GSUB)], sg[b]).wait()

    start_in(0, 0)
    start_in(1, 1)

    @pl.when(sid < 8)
    def _build():
        pltpu.async_copy(w0_hbm, w0_v, sw)
        pltpu.async_copy(w1_hbm, w1_v, sw)
        pltpu.async_copy(w2_hbm, w2_v, sw)
        pltpu.make_async_copy(w0_hbm, w0_v, sw).wait()
        pltpu.make_async_copy(w1_hbm, w1_v, sw).wait()
        pltpu.make_async_copy(w2_hbm, w2_v, sw).wait()

        def build_combo(r, _):
            c = jnp.minimum(8 * sid + r, 59)
            i = c // 12
            rr = c - i * 12
            j = rr // 2
            k = rr - j * 2
            for v in range(8):
                sl = pl.ds(16 * v, 16)
                combo_v[r, sl] = w0_v[i, sl] + w1_v[j, sl] + w2_v[k, sl]
            return _

        lax.fori_loop(0, 8, build_combo, None)
        pltpu.sync_copy(combo_v, combo_sh.at[pl.ds(8 * sid, 8)])

    wait_in(0)
    comp_idx(0)
    plsc.subcore_barrier()
    start_gather(0)

    def pair_body(p, _):
        wait_in(1)
        comp_idx(1)

        @pl.when(p >= 1)
        def _():
            wait_out(1)

        start_gather(1)
        wait_gather(0)
        start_out(2 * p, 0)
        start_in(2 * p + 2, 0)

        wait_in(0)
        comp_idx(0)
        wait_out(0)
        start_gather(0)
        wait_gather(1)
        start_out(2 * p + 1, 1)

        @pl.when(p < 11)
        def _():
            start_in(2 * p + 3, 1)
        return _

    lax.fori_loop(0, (NCHUNK - 1) // 2, pair_body, None)

    wait_gather(0)
    start_out(NCHUNK - 1, 0)
    wait_out(1)
    wait_out(0)


def kernel(edge_attr, W0, W1, W2):
    ea = edge_attr.astype(jnp.int32)
    return _bond_encoder_sc(ea[:, 0], ea[:, 1], ea[:, 2], W0, W1, W2)
